# baseline (speedup 1.0000x reference)
import functools

import jax
import jax.numpy as jnp
from jax import lax
from jax.experimental import pallas as pl
from jax.experimental.pallas import tpu as pltpu
from jax.experimental.pallas import tpu_sc as plsc

_B = 16384
_K = 64
_NW = 32
_RPW = _B // _NW
_CHUNK = 128
_NCHUNK = _RPW // _CHUNK
_GROUPS = _RPW // 16


def _sc_dot(uid_hbm, mid_hbm, u_emb_hbm, i_emb_hbm, out_hbm,
            uidx, midx, urows, vrows, outv, sem):
    wid = lax.axis_index("s") * 2 + lax.axis_index("c")

    pltpu.sync_copy(uid_hbm.at[wid], uidx)
    pltpu.sync_copy(mid_hbm.at[wid], midx)

    copies = []
    for c in range(_NCHUNK):
        dst = pl.ds(c * _CHUNK, _CHUNK)
        copies.append(pltpu.async_copy(u_emb_hbm.at[uidx.at[c]], urows.at[dst], sem))
        copies.append(pltpu.async_copy(i_emb_hbm.at[midx.at[c]], vrows.at[dst], sem))
    for cp in copies:
        cp.wait()

    iota16 = lax.iota(jnp.int32, 16)
    _dnums = lax.GatherDimensionNumbers(
        offset_dims=(), collapsed_slice_dims=(0,), start_index_map=(0,))

    def _shuffle(v, idx):
        return lax.gather(v, idx[:, None], _dnums, slice_sizes=(1,),
                          mode=lax.GatherScatterMode.PROMISE_IN_BOUNDS)

    def group(g, carry):
        outvec = jnp.zeros((16,), jnp.float32)
        for rr in range(16):
            r = g * 16 + rr
            acc = jnp.zeros((16,), jnp.float32)
            for j in range(_K // 16):
                sl = pl.ds(j * 16, 16)
                acc = acc + urows[r, sl] * vrows[r, sl]
            for sh in (8, 4, 2, 1):
                acc = acc + _shuffle(acc, iota16 ^ sh)
            outvec = jnp.where(iota16 == rr, acc, outvec)
        outv[pl.ds(pl.multiple_of(g * 16, 16), 16)] = outvec
        return carry

    lax.fori_loop(0, _GROUPS, group, 0)

    pltpu.sync_copy(outv, out_hbm.at[wid])


@jax.jit
def kernel(uid, mid, u_emb, i_emb):
    mesh = plsc.VectorSubcoreMesh(core_axis_name="c", subcore_axis_name="s")
    fn = functools.partial(
        pl.kernel,
        mesh=mesh,
        out_type=jax.ShapeDtypeStruct((_NW, _RPW), jnp.float32),
        scratch_types=[
            pltpu.VMEM((_NCHUNK, _CHUNK), jnp.int32),
            pltpu.VMEM((_NCHUNK, _CHUNK), jnp.int32),
            pltpu.VMEM((_RPW, _K), jnp.float32),
            pltpu.VMEM((_RPW, _K), jnp.float32),
            pltpu.VMEM((_RPW,), jnp.float32),
            pltpu.SemaphoreType.DMA,
        ],
        compiler_params=pltpu.CompilerParams(use_tc_tiling_on_sc=False),
    )(_sc_dot)
    out = fn(uid.reshape(_NW, _NCHUNK, _CHUNK), mid.reshape(_NW, _NCHUNK, _CHUNK),
             u_emb, i_emb)
    return out.reshape(_B)
